# baseline (speedup 1.0000x reference)
import functools
import jax
import jax.numpy as jnp
from jax import lax
from jax.experimental import pallas as pl
from jax.experimental.pallas import tpu as pltpu
from jax.experimental.pallas import tpu_sc as plsc

VOCAB_PAD = 64
ROWS = 40
E = 1024
TOK_BLK = 256
NBANK = 7 * VOCAB_PAD
NULL_ROW = 4 * VOCAB_PAD + VOCAB_PAD - 1
KTOK = 16


def _prep_body(ids_ref, m_ref, table_ref, pca_ref, wc_ref, wf_ref, bc_ref, bf_ref,
               bank_ref, idxc_ref, idxl_ref, idxr_ref):
    hi = jax.lax.Precision.HIGHEST
    f32 = jnp.float32
    wtop = wf_ref[0:E, :]
    wbot = wf_ref[E:2 * E, :]
    t2 = jnp.dot(table_ref[...], wtop, preferred_element_type=f32, precision=hi)
    pc1 = jnp.dot(pca_ref[...], wc_ref[...], preferred_element_type=f32, precision=hi)
    p2 = jnp.dot(pc1, wbot, preferred_element_type=f32, precision=hi)
    bias = bf_ref[...] + jnp.dot(bc_ref[...], wbot, preferred_element_type=f32, precision=hi)

    bank_ref[...] = jnp.zeros((NBANK, E), f32)
    bank_ref[0:ROWS, :] = t2 + bias
    for ci, sc in ((1, 1.0), (2, 0.5), (3, 1.0 / 3.0)):
        bank_ref[ci * VOCAB_PAD:ci * VOCAB_PAD + ROWS, :] = t2 + p2 * sc + bias
        bank_ref[(3 + ci) * VOCAB_PAD:(3 + ci) * VOCAB_PAD + ROWS, :] = p2 * sc

    ids = ids_ref[...]
    m = m_ref[...]
    nb = ids.shape[0]
    blk_row = jax.lax.broadcasted_iota(jnp.int32, (nb, 1), 0)
    bpr = nb // 4

    def shift_left(a):
        lastcol = a[:, TOK_BLK - 1:TOK_BLK]
        prev = jnp.concatenate([jnp.zeros((1, 1), a.dtype), lastcol[:-1, :]], axis=0)
        prev = jnp.where(blk_row % bpr == 0, jnp.zeros((1, 1), a.dtype), prev)
        return jnp.concatenate([prev, a[:, :TOK_BLK - 1]], axis=1)

    def shift_right(a):
        firstcol = a[:, 0:1]
        nxt = jnp.concatenate([firstcol[1:, :], jnp.zeros((1, 1), a.dtype)], axis=0)
        nxt = jnp.where(blk_row % bpr == bpr - 1, jnp.zeros((1, 1), a.dtype), nxt)
        return jnp.concatenate([a[:, 1:], nxt], axis=1)

    idl2 = shift_left(ids)
    ml2 = shift_left(m)
    idr2 = shift_right(ids)
    mr2 = shift_right(m)
    cc = jnp.clip(ml2 + m + mr2, 1, 3)
    idxc_ref[:, 0, :] = jnp.where(m == 1, cc * VOCAB_PAD + ids, ids)
    idxl_ref[:, 0, :] = jnp.where(ml2 == 1, (3 + cc) * VOCAB_PAD + idl2, NULL_ROW)
    idxr_ref[:, 0, :] = jnp.where(mr2 == 1, (3 + cc) * VOCAB_PAD + idr2, NULL_ROW)


def _lane_sum(v):
    iota = jax.lax.broadcasted_iota(jnp.int32, (16,), 0)
    dnums = lax.GatherDimensionNumbers(
        offset_dims=(), collapsed_slice_dims=(0,), start_index_map=(0,))
    for sh in (8, 4, 2, 1):
        idx = jnp.bitwise_xor(iota, sh)
        v = v + lax.gather(v, idx[:, None], dnums, slice_sizes=(1,),
                           mode=lax.GatherScatterMode.PROMISE_IN_BOUNDS)
    return v


def _sc_rsqrt(x):
    y = jnp.ones_like(x)
    steps = [(64, 2.0**-32, 2.0**32), (32, 2.0**-16, 2.0**16),
             (16, 2.0**-8, 2.0**8), (8, 2.0**-4, 2.0**4),
             (4, 2.0**-2, 2.0**2), (2, 2.0**-1, 2.0**1)]
    for k, dn, up in steps:
        z = x * y * y
        y = y * jnp.where(z >= 2.0**k, dn, 1.0) * jnp.where(z < 2.0**-k, up, 1.0)
    z = x * y * y
    y = y * jnp.where(z >= 2.0, 0.70710678, 1.0) * jnp.where(z < 0.5, 1.41421356, 1.0)
    for _ in range(5):
        y = y * (1.5 - 0.5 * x * y * y)
    return y


def _sc_body(bank_hbm, idxc_hbm, idxl_hbm, idxr_hbm, g_hbm, b_hbm, out_hbm,
             idxc_v, idxl_v, idxr_v, c_buf, l_buf, r_buf, out_buf, g_v, b_v, sem):
    info = plsc.get_sparse_core_info()
    nc = info.num_cores
    wid = lax.axis_index("s") * nc + lax.axis_index("c")
    base = wid * TOK_BLK
    pltpu.sync_copy(g_hbm, g_v)
    pltpu.sync_copy(b_hbm, b_v)

    def blk_body(blk, carry):
        off = base + blk * KTOK
        pltpu.sync_copy(idxc_hbm.at[pl.ds(off, KTOK)], idxc_v)
        pltpu.sync_copy(idxl_hbm.at[pl.ds(off, KTOK)], idxl_v)
        pltpu.sync_copy(idxr_hbm.at[pl.ds(off, KTOK)], idxr_v)
        cpc = pltpu.async_copy(bank_hbm.at[idxc_v], c_buf, sem)
        cpl = pltpu.async_copy(bank_hbm.at[idxl_v], l_buf, sem)
        cpr = pltpu.async_copy(bank_hbm.at[idxr_v], r_buf, sem)
        cpc.wait()
        cpl.wait()
        cpr.wait()

        def tok_body(t, carry2):
            s = jnp.zeros((16,), jnp.float32)
            q = jnp.zeros((16,), jnp.float32)
            for j in range(E // 16):
                dsj = pl.ds(16 * j, 16)
                v = c_buf[t, dsj] + l_buf[t, dsj] + r_buf[t, dsj]
                out_buf[t, dsj] = v
                s = s + v
                q = q + v * v
            mu = _lane_sum(s) * (1.0 / E)
            var = jnp.maximum(_lane_sum(q) * (1.0 / E) - mu * mu, 0.0)
            rv = _sc_rsqrt(var + 1e-5)
            for j in range(E // 16):
                dsj = pl.ds(16 * j, 16)
                out_buf[t, dsj] = (out_buf[t, dsj] - mu) * rv * g_v[dsj] + b_v[dsj]
            return carry2

        lax.fori_loop(0, KTOK, tok_body, 0)
        pltpu.sync_copy(out_buf, out_hbm.at[pl.ds(off, KTOK)])
        return carry

    lax.fori_loop(0, TOK_BLK // KTOK, blk_body, 0)


def kernel(input_ids, attention_mask, table, pca_lookup, W_cont, b_cont,
           W_fuse, b_fuse, gamma, beta):
    B, S = input_ids.shape
    N = B * S
    nblk = N // TOK_BLK
    f32 = jnp.float32

    table_p = jnp.zeros((ROWS, E), f32).at[:table.shape[0]].set(table)
    pca_p = jnp.zeros((ROWS, 128), f32).at[:pca_lookup.shape[0], :pca_lookup.shape[1]].set(pca_lookup)
    wc_p = jnp.zeros((128, E), f32).at[:W_cont.shape[0]].set(W_cont)
    ids32 = input_ids.astype(jnp.int32).reshape(nblk, TOK_BLK)
    m32 = attention_mask.astype(jnp.int32).reshape(nblk, TOK_BLK)

    i3 = jax.ShapeDtypeStruct((nblk, 1, TOK_BLK), jnp.int32)
    bank, idxc, idxl, idxr = pl.pallas_call(
        _prep_body,
        out_shape=(jax.ShapeDtypeStruct((NBANK, E), f32), i3, i3, i3),
    )(ids32, m32, table_p, pca_p, wc_p, W_fuse,
      b_cont.reshape(1, E), b_fuse.reshape(1, E))

    mesh = plsc.VectorSubcoreMesh(core_axis_name="c", subcore_axis_name="s")
    sc = functools.partial(
        pl.kernel,
        mesh=mesh,
        out_type=jax.ShapeDtypeStruct((N, E), f32),
        scratch_types=[
            pltpu.VMEM((KTOK,), jnp.int32),
            pltpu.VMEM((KTOK,), jnp.int32),
            pltpu.VMEM((KTOK,), jnp.int32),
            pltpu.VMEM((KTOK, E), f32),
            pltpu.VMEM((KTOK, E), f32),
            pltpu.VMEM((KTOK, E), f32),
            pltpu.VMEM((KTOK, E), f32),
            pltpu.VMEM((E,), f32),
            pltpu.VMEM((E,), f32),
            pltpu.SemaphoreType.DMA,
        ],
    )(_sc_body)
    out = sc(bank, idxc.reshape(N), idxl.reshape(N), idxr.reshape(N), gamma, beta)
    return out.reshape(B, S, E)


# speedup vs baseline: 1.2429x; 1.2429x over previous
import functools
import jax
import jax.numpy as jnp
from jax import lax
from jax.experimental import pallas as pl
from jax.experimental.pallas import tpu as pltpu
from jax.experimental.pallas import tpu_sc as plsc

VOCAB_PAD = 64
ROWS = 40
E = 1024
TOK_BLK = 256
NBANK = 7 * VOCAB_PAD
NULL_ROW = 4 * VOCAB_PAD + VOCAB_PAD - 1
KTOK = 16


def _prep_body(ids_ref, m_ref, table_ref, pca_ref, wc_ref, wf_ref, bc_ref, bf_ref,
               bank_ref, idxc_ref, idxl_ref, idxr_ref):
    hi = jax.lax.Precision.HIGHEST
    f32 = jnp.float32
    wtop = wf_ref[0:E, :]
    wbot = wf_ref[E:2 * E, :]
    t2 = jnp.dot(table_ref[...], wtop, preferred_element_type=f32, precision=hi)
    pc1 = jnp.dot(pca_ref[...], wc_ref[...], preferred_element_type=f32, precision=hi)
    p2 = jnp.dot(pc1, wbot, preferred_element_type=f32, precision=hi)
    bias = bf_ref[...] + jnp.dot(bc_ref[...], wbot, preferred_element_type=f32, precision=hi)

    bank_ref[...] = jnp.zeros((NBANK, E), f32)
    bank_ref[0:ROWS, :] = t2 + bias
    for ci, sc in ((1, 1.0), (2, 0.5), (3, 1.0 / 3.0)):
        bank_ref[ci * VOCAB_PAD:ci * VOCAB_PAD + ROWS, :] = t2 + p2 * sc + bias
        bank_ref[(3 + ci) * VOCAB_PAD:(3 + ci) * VOCAB_PAD + ROWS, :] = p2 * sc

    ids = ids_ref[...]
    m = m_ref[...]
    nb = ids.shape[0]
    blk_row = jax.lax.broadcasted_iota(jnp.int32, (nb, 1), 0)
    bpr = nb // 4

    def shift_left(a):
        lastcol = a[:, TOK_BLK - 1:TOK_BLK]
        prev = jnp.concatenate([jnp.zeros((1, 1), a.dtype), lastcol[:-1, :]], axis=0)
        prev = jnp.where(blk_row % bpr == 0, jnp.zeros((1, 1), a.dtype), prev)
        return jnp.concatenate([prev, a[:, :TOK_BLK - 1]], axis=1)

    def shift_right(a):
        firstcol = a[:, 0:1]
        nxt = jnp.concatenate([firstcol[1:, :], jnp.zeros((1, 1), a.dtype)], axis=0)
        nxt = jnp.where(blk_row % bpr == bpr - 1, jnp.zeros((1, 1), a.dtype), nxt)
        return jnp.concatenate([a[:, 1:], nxt], axis=1)

    idl2 = shift_left(ids)
    ml2 = shift_left(m)
    idr2 = shift_right(ids)
    mr2 = shift_right(m)
    cc = jnp.clip(ml2 + m + mr2, 1, 3)
    idxc_ref[:, 0, :] = jnp.where(m == 1, cc * VOCAB_PAD + ids, ids)
    idxl_ref[:, 0, :] = jnp.where(ml2 == 1, (3 + cc) * VOCAB_PAD + idl2, NULL_ROW)
    idxr_ref[:, 0, :] = jnp.where(mr2 == 1, (3 + cc) * VOCAB_PAD + idr2, NULL_ROW)


def _lane_sum(v):
    iota = jax.lax.broadcasted_iota(jnp.int32, (16,), 0)
    dnums = lax.GatherDimensionNumbers(
        offset_dims=(), collapsed_slice_dims=(0,), start_index_map=(0,))
    for sh in (8, 4, 2, 1):
        idx = jnp.bitwise_xor(iota, sh)
        v = v + lax.gather(v, idx[:, None], dnums, slice_sizes=(1,),
                           mode=lax.GatherScatterMode.PROMISE_IN_BOUNDS)
    return v


def _sc_rsqrt(x):
    y = jnp.ones_like(x)
    steps = [(64, 2.0**-32, 2.0**32), (32, 2.0**-16, 2.0**16),
             (16, 2.0**-8, 2.0**8), (8, 2.0**-4, 2.0**4),
             (4, 2.0**-2, 2.0**2), (2, 2.0**-1, 2.0**1)]
    for k, dn, up in steps:
        z = x * y * y
        y = y * jnp.where(z >= 2.0**k, dn, 1.0) * jnp.where(z < 2.0**-k, up, 1.0)
    z = x * y * y
    y = y * jnp.where(z >= 2.0, 0.70710678, 1.0) * jnp.where(z < 0.5, 1.41421356, 1.0)
    for _ in range(5):
        y = y * (1.5 - 0.5 * x * y * y)
    return y


def _sc_body(bank_hbm, idxc_hbm, idxl_hbm, idxr_hbm, g_hbm, b_hbm, out_hbm,
             idxc_a, idxl_a, idxr_a, c0, l0, r0, c1, l1, r1,
             out_buf, g_v, b_v, sem0, sem1):
    info = plsc.get_sparse_core_info()
    nc = info.num_cores
    wid = lax.axis_index("s") * nc + lax.axis_index("c")
    base = wid * TOK_BLK
    nblk_w = TOK_BLK // KTOK
    pltpu.sync_copy(g_hbm, g_v)
    pltpu.sync_copy(b_hbm, b_v)
    pltpu.sync_copy(idxc_hbm.at[pl.ds(base, TOK_BLK)], idxc_a)
    pltpu.sync_copy(idxl_hbm.at[pl.ds(base, TOK_BLK)], idxl_a)
    pltpu.sync_copy(idxr_hbm.at[pl.ds(base, TOK_BLK)], idxr_a)

    def start_gathers(blk, bufs, sem):
        for idx_a, buf in zip((idxc_a, idxl_a, idxr_a), bufs):
            idx_vec = idx_a[pl.ds(blk * KTOK, KTOK)]
            pltpu.async_copy(bank_hbm.at[idx_vec], buf, sem)

    def wait_gathers(bufs, sem):
        for buf in bufs:
            pltpu.make_async_copy(bank_hbm.at[pl.ds(0, KTOK)], buf, sem).wait()

    def compute_block(blk, bufs):
        cb, lb, rb = bufs

        def tok_body(t, carry2):
            s = jnp.zeros((16,), jnp.float32)
            q = jnp.zeros((16,), jnp.float32)
            for j in range(E // 16):
                dsj = pl.ds(16 * j, 16)
                v = cb[t, dsj] + lb[t, dsj] + rb[t, dsj]
                out_buf[t, dsj] = v
                s = s + v
                q = q + v * v
            mu = _lane_sum(s) * (1.0 / E)
            var = jnp.maximum(_lane_sum(q) * (1.0 / E) - mu * mu, 0.0)
            rv = _sc_rsqrt(var + 1e-5)
            for j in range(E // 16):
                dsj = pl.ds(16 * j, 16)
                out_buf[t, dsj] = (out_buf[t, dsj] - mu) * rv * g_v[dsj] + b_v[dsj]
            return carry2

        lax.fori_loop(0, KTOK, tok_body, 0)
        pltpu.sync_copy(out_buf, out_hbm.at[pl.ds(base + blk * KTOK, KTOK)])

    bufs0 = (c0, l0, r0)
    bufs1 = (c1, l1, r1)
    start_gathers(0, bufs0, sem0)

    def pair_body(p, carry):
        blk = 2 * p
        start_gathers(blk + 1, bufs1, sem1)
        wait_gathers(bufs0, sem0)
        compute_block(blk, bufs0)

        @pl.when(p < nblk_w // 2 - 1)
        def _():
            start_gathers(blk + 2, bufs0, sem0)

        wait_gathers(bufs1, sem1)
        compute_block(blk + 1, bufs1)
        return carry

    lax.fori_loop(0, nblk_w // 2, pair_body, 0)


def kernel(input_ids, attention_mask, table, pca_lookup, W_cont, b_cont,
           W_fuse, b_fuse, gamma, beta):
    B, S = input_ids.shape
    N = B * S
    nblk = N // TOK_BLK
    f32 = jnp.float32

    table_p = jnp.zeros((ROWS, E), f32).at[:table.shape[0]].set(table)
    pca_p = jnp.zeros((ROWS, 128), f32).at[:pca_lookup.shape[0], :pca_lookup.shape[1]].set(pca_lookup)
    wc_p = jnp.zeros((128, E), f32).at[:W_cont.shape[0]].set(W_cont)
    ids32 = input_ids.astype(jnp.int32).reshape(nblk, TOK_BLK)
    m32 = attention_mask.astype(jnp.int32).reshape(nblk, TOK_BLK)

    i3 = jax.ShapeDtypeStruct((nblk, 1, TOK_BLK), jnp.int32)
    bank, idxc, idxl, idxr = pl.pallas_call(
        _prep_body,
        out_shape=(jax.ShapeDtypeStruct((NBANK, E), f32), i3, i3, i3),
    )(ids32, m32, table_p, pca_p, wc_p, W_fuse,
      b_cont.reshape(1, E), b_fuse.reshape(1, E))

    mesh = plsc.VectorSubcoreMesh(core_axis_name="c", subcore_axis_name="s")
    sc = functools.partial(
        pl.kernel,
        mesh=mesh,
        out_type=jax.ShapeDtypeStruct((N, E), f32),
        scratch_types=(
            [pltpu.VMEM((TOK_BLK,), jnp.int32)] * 3
            + [pltpu.VMEM((KTOK, E), f32)] * 7
            + [pltpu.VMEM((E,), f32)] * 2
            + [pltpu.SemaphoreType.DMA] * 2
        ),
    )(_sc_body)
    out = sc(bank, idxc.reshape(N), idxl.reshape(N), idxr.reshape(N), gamma, beta)
    return out.reshape(B, S, E)


# speedup vs baseline: 2.8357x; 2.2816x over previous
import functools
import jax
import jax.numpy as jnp
from jax import lax
from jax.experimental import pallas as pl
from jax.experimental.pallas import tpu as pltpu
from jax.experimental.pallas import tpu_sc as plsc

VOCAB_PAD = 64
ROWS = 40
E = 1024
TOK_BLK = 256
NBANK = 7 * VOCAB_PAD
NULL_ROW = 4 * VOCAB_PAD + VOCAB_PAD - 1
KTOK = 16
SC_BLKS = 4
N_WORKERS = 32


def _prep_body(ids_ref, m_ref, table_ref, pca_ref, wc_ref, wf_ref, bc_ref, bf_ref,
               t2_ref, p2_ref, bias_ref, idc_ref, idcp_ref, idl_ref, idr_ref, inv_ref,
               bank_ref, sidxc_ref, sidxl_ref, sidxr_ref):
    hi = jax.lax.Precision.HIGHEST
    f32 = jnp.float32
    wtop = wf_ref[0:E, :]
    wbot = wf_ref[E:2 * E, :]
    t2 = jnp.dot(table_ref[...], wtop, preferred_element_type=f32, precision=hi)
    pc1 = jnp.dot(pca_ref[...], wc_ref[...], preferred_element_type=f32, precision=hi)
    p2 = jnp.dot(pc1, wbot, preferred_element_type=f32, precision=hi)
    bias = bf_ref[...] + jnp.dot(bc_ref[...], wbot, preferred_element_type=f32, precision=hi)

    def hilo(x, out_ref):
        xh = x.astype(jnp.bfloat16)
        xl = (x - xh.astype(f32)).astype(jnp.bfloat16)
        out_ref[...] = jnp.zeros((2 * VOCAB_PAD, E), jnp.bfloat16)
        out_ref[0:ROWS, :] = xh
        out_ref[VOCAB_PAD:VOCAB_PAD + ROWS, :] = xl

    hilo(t2, t2_ref)
    hilo(p2, p2_ref)
    bias_ref[...] = bias

    bank_ref[...] = jnp.zeros((NBANK, E), f32)
    bank_ref[0:ROWS, :] = t2 + bias
    for ci, sc in ((1, 1.0), (2, 0.5), (3, 1.0 / 3.0)):
        bank_ref[ci * VOCAB_PAD:ci * VOCAB_PAD + ROWS, :] = t2 + p2 * sc + bias
        bank_ref[(3 + ci) * VOCAB_PAD:(3 + ci) * VOCAB_PAD + ROWS, :] = p2 * sc

    ids = ids_ref[...]
    m = m_ref[...]
    nb = ids.shape[0]
    blk_row = jax.lax.broadcasted_iota(jnp.int32, (nb, 1), 0)
    bpr = nb // 4

    def shift_left(a):
        lastcol = a[:, TOK_BLK - 1:TOK_BLK]
        prev = jnp.concatenate([jnp.zeros((1, 1), a.dtype), lastcol[:-1, :]], axis=0)
        prev = jnp.where(blk_row % bpr == 0, jnp.zeros((1, 1), a.dtype), prev)
        return jnp.concatenate([prev, a[:, :TOK_BLK - 1]], axis=1)

    def shift_right(a):
        firstcol = a[:, 0:1]
        nxt = jnp.concatenate([firstcol[1:, :], jnp.zeros((1, 1), a.dtype)], axis=0)
        nxt = jnp.where(blk_row % bpr == bpr - 1, jnp.zeros((1, 1), a.dtype), nxt)
        return jnp.concatenate([a[:, 1:], nxt], axis=1)

    idl2 = shift_left(ids)
    ml2 = shift_left(m)
    idr2 = shift_right(ids)
    mr2 = shift_right(m)
    cc = jnp.clip(ml2 + m + mr2, 1, 3)
    zrow = VOCAB_PAD - 1
    idc_ref[:, 0, :] = ids
    idcp_ref[:, 0, :] = jnp.where(m == 1, ids, zrow)
    idl_ref[:, 0, :] = jnp.where(ml2 == 1, idl2, zrow)
    idr_ref[:, 0, :] = jnp.where(mr2 == 1, idr2, zrow)
    inv_ref[:, 0, :] = 1.0 / cc.astype(f32)
    sidxc_ref[:, 0, :] = jnp.where(m == 1, cc * VOCAB_PAD + ids, ids)
    sidxl_ref[:, 0, :] = jnp.where(ml2 == 1, (3 + cc) * VOCAB_PAD + idl2, NULL_ROW)
    sidxr_ref[:, 0, :] = jnp.where(mr2 == 1, (3 + cc) * VOCAB_PAD + idr2, NULL_ROW)


def _tc_body(idc_ref, idcp_ref, idl_ref, idr_ref, inv_ref,
             t2_ref, p2_ref, bias_ref, g_ref, b_ref, out_ref):
    n = TOK_BLK
    iota = jax.lax.broadcasted_iota(jnp.int32, (1, VOCAB_PAD), 1)

    def onehot(ids3):
        ids = ids3[...].reshape(n, 1)
        return (ids == iota).astype(jnp.float32)

    oh_c = onehot(idc_ref).astype(jnp.bfloat16)
    oh_w = (onehot(idcp_ref) + onehot(idl_ref) + onehot(idr_ref)).astype(jnp.bfloat16)
    oh_c2 = jnp.concatenate([oh_c, oh_c], axis=1)
    oh_w2 = jnp.concatenate([oh_w, oh_w], axis=1)
    d1 = jnp.dot(oh_c2, t2_ref[...], preferred_element_type=jnp.float32)
    d2 = jnp.dot(oh_w2, p2_ref[...], preferred_element_type=jnp.float32)
    inv = inv_ref[...].reshape(n, 1)
    fused = d1 + d2 * inv + bias_ref[...]
    mu = jnp.mean(fused, axis=-1, keepdims=True)
    d = fused - mu
    var = jnp.mean(d * d, axis=-1, keepdims=True)
    out_ref[...] = d * jax.lax.rsqrt(var + 1e-5) * g_ref[...] + b_ref[...]


def _lane_sum(v):
    iota = jax.lax.broadcasted_iota(jnp.int32, (16,), 0)
    dnums = lax.GatherDimensionNumbers(
        offset_dims=(), collapsed_slice_dims=(0,), start_index_map=(0,))
    for sh in (8, 4, 2, 1):
        idx = jnp.bitwise_xor(iota, sh)
        v = v + lax.gather(v, idx[:, None], dnums, slice_sizes=(1,),
                           mode=lax.GatherScatterMode.PROMISE_IN_BOUNDS)
    return v


def _sc_rsqrt(x):
    y = jnp.ones_like(x)
    steps = [(64, 2.0**-32, 2.0**32), (32, 2.0**-16, 2.0**16),
             (16, 2.0**-8, 2.0**8), (8, 2.0**-4, 2.0**4),
             (4, 2.0**-2, 2.0**2), (2, 2.0**-1, 2.0**1)]
    for k, dn, up in steps:
        z = x * y * y
        y = y * jnp.where(z >= 2.0**k, dn, 1.0) * jnp.where(z < 2.0**-k, up, 1.0)
    z = x * y * y
    y = y * jnp.where(z >= 2.0, 0.70710678, 1.0) * jnp.where(z < 0.5, 1.41421356, 1.0)
    for _ in range(5):
        y = y * (1.5 - 0.5 * x * y * y)
    return y


def _make_sc_body(tok_offset, tok_per_worker):
    nblk_w = tok_per_worker // KTOK

    def _sc_body(bank_hbm, idxc_hbm, idxl_hbm, idxr_hbm, g_hbm, b_hbm, out_hbm,
                 idxc_a, idxl_a, idxr_a, c0, l0, r0, c1, l1, r1,
                 out_buf, g_v, b_v, sem0, sem1):
        info = plsc.get_sparse_core_info()
        nc = info.num_cores
        wid = lax.axis_index("s") * nc + lax.axis_index("c")
        obase = wid * tok_per_worker
        ibase = tok_offset + obase
        pltpu.sync_copy(g_hbm, g_v)
        pltpu.sync_copy(b_hbm, b_v)
        pltpu.sync_copy(idxc_hbm.at[pl.ds(ibase, tok_per_worker)], idxc_a)
        pltpu.sync_copy(idxl_hbm.at[pl.ds(ibase, tok_per_worker)], idxl_a)
        pltpu.sync_copy(idxr_hbm.at[pl.ds(ibase, tok_per_worker)], idxr_a)

        def start_gathers(blk, bufs, sem):
            for idx_a, buf in zip((idxc_a, idxl_a, idxr_a), bufs):
                idx_vec = idx_a[pl.ds(blk * KTOK, KTOK)]
                pltpu.async_copy(bank_hbm.at[idx_vec], buf, sem)

        def wait_gathers(bufs, sem):
            for buf in bufs:
                pltpu.make_async_copy(bank_hbm.at[pl.ds(0, KTOK)], buf, sem).wait()

        def compute_block(blk, bufs):
            cb, lb, rb = bufs

            def tok_body(t, carry2):
                s = jnp.zeros((16,), jnp.float32)
                q = jnp.zeros((16,), jnp.float32)
                for j in range(E // 16):
                    dsj = pl.ds(16 * j, 16)
                    v = cb[t, dsj] + lb[t, dsj] + rb[t, dsj]
                    out_buf[t, dsj] = v
                    s = s + v
                    q = q + v * v
                mu = _lane_sum(s) * (1.0 / E)
                var = jnp.maximum(_lane_sum(q) * (1.0 / E) - mu * mu, 0.0)
                rv = _sc_rsqrt(var + 1e-5)
                for j in range(E // 16):
                    dsj = pl.ds(16 * j, 16)
                    out_buf[t, dsj] = (out_buf[t, dsj] - mu) * rv * g_v[dsj] + b_v[dsj]
                return carry2

            lax.fori_loop(0, KTOK, tok_body, 0)
            pltpu.sync_copy(out_buf, out_hbm.at[pl.ds(obase + blk * KTOK, KTOK)])

        bufs0 = (c0, l0, r0)
        bufs1 = (c1, l1, r1)
        start_gathers(0, bufs0, sem0)

        def pair_body(p, carry):
            blk = 2 * p
            start_gathers(blk + 1, bufs1, sem1)
            wait_gathers(bufs0, sem0)
            compute_block(blk, bufs0)

            @pl.when(p < nblk_w // 2 - 1)
            def _():
                start_gathers(blk + 2, bufs0, sem0)

            wait_gathers(bufs1, sem1)
            compute_block(blk + 1, bufs1)
            return carry

        lax.fori_loop(0, nblk_w // 2, pair_body, 0)

    return _sc_body


def kernel(input_ids, attention_mask, table, pca_lookup, W_cont, b_cont,
           W_fuse, b_fuse, gamma, beta):
    B, S = input_ids.shape
    N = B * S
    nblk = N // TOK_BLK
    nblk_tc = nblk - SC_BLKS
    n_sc = SC_BLKS * TOK_BLK
    tok_pw = n_sc // N_WORKERS
    f32 = jnp.float32

    table_p = jnp.zeros((ROWS, E), f32).at[:table.shape[0]].set(table)
    pca_p = jnp.zeros((ROWS, 128), f32).at[:pca_lookup.shape[0], :pca_lookup.shape[1]].set(pca_lookup)
    wc_p = jnp.zeros((128, E), f32).at[:W_cont.shape[0]].set(W_cont)
    ids32 = input_ids.astype(jnp.int32).reshape(nblk, TOK_BLK)
    m32 = attention_mask.astype(jnp.int32).reshape(nblk, TOK_BLK)

    i3 = jax.ShapeDtypeStruct((nblk, 1, TOK_BLK), jnp.int32)
    (t2c, p2c, bias, idc, idcp, idl, idr, inv,
     bank, sidxc, sidxl, sidxr) = pl.pallas_call(
        _prep_body,
        out_shape=(
            jax.ShapeDtypeStruct((2 * VOCAB_PAD, E), jnp.bfloat16),
            jax.ShapeDtypeStruct((2 * VOCAB_PAD, E), jnp.bfloat16),
            jax.ShapeDtypeStruct((1, E), f32),
            i3, i3, i3, i3,
            jax.ShapeDtypeStruct((nblk, 1, TOK_BLK), f32),
            jax.ShapeDtypeStruct((NBANK, E), f32),
            i3, i3, i3,
        ),
    )(ids32, m32, table_p, pca_p, wc_p, W_fuse,
      b_cont.reshape(1, E), b_fuse.reshape(1, E))

    sc_call = functools.partial(
        pl.kernel,
        mesh=plsc.VectorSubcoreMesh(core_axis_name="c", subcore_axis_name="s"),
        out_type=jax.ShapeDtypeStruct((n_sc, E), f32),
        scratch_types=(
            [pltpu.VMEM((tok_pw,), jnp.int32)] * 3
            + [pltpu.VMEM((KTOK, E), f32)] * 7
            + [pltpu.VMEM((E,), f32)] * 2
            + [pltpu.SemaphoreType.DMA] * 2
        ),
    )(_make_sc_body(nblk_tc * TOK_BLK, tok_pw))
    sc_out = sc_call(bank, sidxc.reshape(N), sidxl.reshape(N), sidxr.reshape(N),
                     gamma, beta)

    blk_i = pl.BlockSpec((1, 1, TOK_BLK), lambda i: (i, 0, 0))
    full = lambda shape: pl.BlockSpec(shape, lambda i: (0,) * len(shape))
    tc_out = pl.pallas_call(
        _tc_body,
        grid=(nblk_tc,),
        in_specs=[blk_i, blk_i, blk_i, blk_i, blk_i,
                  full((2 * VOCAB_PAD, E)), full((2 * VOCAB_PAD, E)), full((1, E)),
                  full((1, E)), full((1, E))],
        out_specs=pl.BlockSpec((TOK_BLK, E), lambda i: (i, 0)),
        out_shape=jax.ShapeDtypeStruct((nblk_tc * TOK_BLK, E), f32),
    )(idc, idcp, idl, idr, inv, t2c, p2c, bias,
      gamma.reshape(1, E), beta.reshape(1, E))

    out = jnp.concatenate([tc_out, sc_out], axis=0)
    return out.reshape(B, S, E)


# speedup vs baseline: 3.1539x; 1.1122x over previous
import functools
import jax
import jax.numpy as jnp
from jax import lax
from jax.experimental import pallas as pl
from jax.experimental.pallas import tpu as pltpu
from jax.experimental.pallas import tpu_sc as plsc

VOCAB_PAD = 64
ROWS = 40
E = 1024
TOK_BLK = 256
NBANK = 7 * VOCAB_PAD
NULL_ROW = 4 * VOCAB_PAD + VOCAB_PAD - 1
KTOK = 16
SC_BLKS = 2
N_WORKERS = 32


def _prep_body(ids_ref, m_ref, table_ref, pca_ref, wc_ref, wf_ref, bc_ref, bf_ref,
               t2_ref, p2_ref, bias_ref, idc_ref, idcp_ref, idl_ref, idr_ref, inv_ref,
               bank_ref, sidxc_ref, sidxl_ref, sidxr_ref):
    hi = jax.lax.Precision.HIGHEST
    f32 = jnp.float32
    wtop = wf_ref[0:E, :]
    wbot = wf_ref[E:2 * E, :]
    t2 = jnp.dot(table_ref[...], wtop, preferred_element_type=f32, precision=hi)
    pc1 = jnp.dot(pca_ref[...], wc_ref[...], preferred_element_type=f32, precision=hi)
    p2 = jnp.dot(pc1, wbot, preferred_element_type=f32, precision=hi)
    bias = bf_ref[...] + jnp.dot(bc_ref[...], wbot, preferred_element_type=f32, precision=hi)

    def hilo(x, out_ref):
        xh = x.astype(jnp.bfloat16)
        xl = (x - xh.astype(f32)).astype(jnp.bfloat16)
        out_ref[...] = jnp.zeros((2 * VOCAB_PAD, E), jnp.bfloat16)
        out_ref[0:ROWS, :] = xh
        out_ref[VOCAB_PAD:VOCAB_PAD + ROWS, :] = xl

    hilo(t2, t2_ref)
    hilo(p2, p2_ref)
    bias_ref[...] = bias

    bank_ref[...] = jnp.zeros((NBANK, E), f32)
    bank_ref[0:ROWS, :] = t2 + bias
    for ci, sc in ((1, 1.0), (2, 0.5), (3, 1.0 / 3.0)):
        bank_ref[ci * VOCAB_PAD:ci * VOCAB_PAD + ROWS, :] = t2 + p2 * sc + bias
        bank_ref[(3 + ci) * VOCAB_PAD:(3 + ci) * VOCAB_PAD + ROWS, :] = p2 * sc

    ids = ids_ref[...]
    m = m_ref[...]
    nb = ids.shape[0]
    blk_row = jax.lax.broadcasted_iota(jnp.int32, (nb, 1), 0)
    bpr = nb // 4

    def shift_left(a):
        lastcol = a[:, TOK_BLK - 1:TOK_BLK]
        prev = jnp.concatenate([jnp.zeros((1, 1), a.dtype), lastcol[:-1, :]], axis=0)
        prev = jnp.where(blk_row % bpr == 0, jnp.zeros((1, 1), a.dtype), prev)
        return jnp.concatenate([prev, a[:, :TOK_BLK - 1]], axis=1)

    def shift_right(a):
        firstcol = a[:, 0:1]
        nxt = jnp.concatenate([firstcol[1:, :], jnp.zeros((1, 1), a.dtype)], axis=0)
        nxt = jnp.where(blk_row % bpr == bpr - 1, jnp.zeros((1, 1), a.dtype), nxt)
        return jnp.concatenate([a[:, 1:], nxt], axis=1)

    idl2 = shift_left(ids)
    ml2 = shift_left(m)
    idr2 = shift_right(ids)
    mr2 = shift_right(m)
    cc = jnp.clip(ml2 + m + mr2, 1, 3)
    zrow = VOCAB_PAD - 1
    idc_ref[:, 0, :] = ids
    idcp_ref[:, 0, :] = jnp.where(m == 1, ids, zrow)
    idl_ref[:, 0, :] = jnp.where(ml2 == 1, idl2, zrow)
    idr_ref[:, 0, :] = jnp.where(mr2 == 1, idr2, zrow)
    inv_ref[:, 0, :] = 1.0 / cc.astype(f32)
    sidxc_ref[:, 0, :] = jnp.where(m == 1, cc * VOCAB_PAD + ids, ids)
    sidxl_ref[:, 0, :] = jnp.where(ml2 == 1, (3 + cc) * VOCAB_PAD + idl2, NULL_ROW)
    sidxr_ref[:, 0, :] = jnp.where(mr2 == 1, (3 + cc) * VOCAB_PAD + idr2, NULL_ROW)


def _tc_body(idc_ref, idcp_ref, idl_ref, idr_ref, inv_ref,
             t2_ref, p2_ref, bias_ref, g_ref, b_ref, out_ref):
    n = TOK_BLK
    iota = jax.lax.broadcasted_iota(jnp.int32, (1, VOCAB_PAD), 1)

    def onehot(ids3):
        ids = ids3[...].reshape(n, 1)
        return (ids == iota).astype(jnp.float32)

    oh_c = onehot(idc_ref).astype(jnp.bfloat16)
    oh_w = (onehot(idcp_ref) + onehot(idl_ref) + onehot(idr_ref)).astype(jnp.bfloat16)
    oh_c2 = jnp.concatenate([oh_c, oh_c], axis=1)
    oh_w2 = jnp.concatenate([oh_w, oh_w], axis=1)
    d1 = jnp.dot(oh_c2, t2_ref[...], preferred_element_type=jnp.float32)
    d2 = jnp.dot(oh_w2, p2_ref[...], preferred_element_type=jnp.float32)
    inv = inv_ref[...].reshape(n, 1)
    fused = d1 + d2 * inv + bias_ref[...]
    mu = jnp.mean(fused, axis=-1, keepdims=True)
    d = fused - mu
    var = jnp.mean(d * d, axis=-1, keepdims=True)
    out_ref[...] = d * jax.lax.rsqrt(var + 1e-5) * g_ref[...] + b_ref[...]


def _lane_sum(v):
    iota = jax.lax.broadcasted_iota(jnp.int32, (16,), 0)
    dnums = lax.GatherDimensionNumbers(
        offset_dims=(), collapsed_slice_dims=(0,), start_index_map=(0,))
    for sh in (8, 4, 2, 1):
        idx = jnp.bitwise_xor(iota, sh)
        v = v + lax.gather(v, idx[:, None], dnums, slice_sizes=(1,),
                           mode=lax.GatherScatterMode.PROMISE_IN_BOUNDS)
    return v


def _sc_rsqrt(x):
    y = jnp.ones_like(x)
    steps = [(64, 2.0**-32, 2.0**32), (32, 2.0**-16, 2.0**16),
             (16, 2.0**-8, 2.0**8), (8, 2.0**-4, 2.0**4),
             (4, 2.0**-2, 2.0**2), (2, 2.0**-1, 2.0**1)]
    for k, dn, up in steps:
        z = x * y * y
        y = y * jnp.where(z >= 2.0**k, dn, 1.0) * jnp.where(z < 2.0**-k, up, 1.0)
    z = x * y * y
    y = y * jnp.where(z >= 2.0, 0.70710678, 1.0) * jnp.where(z < 0.5, 1.41421356, 1.0)
    for _ in range(5):
        y = y * (1.5 - 0.5 * x * y * y)
    return y


def _make_sc_body(tok_offset, tok_per_worker):
    nblk_w = tok_per_worker // KTOK

    def _sc_body(bank_hbm, idxc_hbm, idxl_hbm, idxr_hbm, g_hbm, b_hbm, out_hbm,
                 idxc_a, idxl_a, idxr_a, c0, l0, r0, c1, l1, r1,
                 out_buf, g_v, b_v, sem0, sem1):
        info = plsc.get_sparse_core_info()
        nc = info.num_cores
        wid = lax.axis_index("s") * nc + lax.axis_index("c")
        obase = wid * tok_per_worker
        ibase = tok_offset + obase
        pltpu.sync_copy(g_hbm, g_v)
        pltpu.sync_copy(b_hbm, b_v)
        pltpu.sync_copy(idxc_hbm.at[pl.ds(ibase, tok_per_worker)], idxc_a)
        pltpu.sync_copy(idxl_hbm.at[pl.ds(ibase, tok_per_worker)], idxl_a)
        pltpu.sync_copy(idxr_hbm.at[pl.ds(ibase, tok_per_worker)], idxr_a)

        def start_gathers(blk, bufs, sem):
            for idx_a, buf in zip((idxc_a, idxl_a, idxr_a), bufs):
                idx_vec = idx_a[pl.ds(blk * KTOK, KTOK)]
                pltpu.async_copy(bank_hbm.at[idx_vec], buf, sem)

        def wait_gathers(bufs, sem):
            for buf in bufs:
                pltpu.make_async_copy(bank_hbm.at[pl.ds(0, KTOK)], buf, sem).wait()

        def compute_block(blk, bufs):
            cb, lb, rb = bufs

            def tok_body(t, carry2):
                s = jnp.zeros((16,), jnp.float32)
                q = jnp.zeros((16,), jnp.float32)
                for j in range(E // 16):
                    dsj = pl.ds(16 * j, 16)
                    v = cb[t, dsj] + lb[t, dsj] + rb[t, dsj]
                    out_buf[t, dsj] = v
                    s = s + v
                    q = q + v * v
                mu = _lane_sum(s) * (1.0 / E)
                var = jnp.maximum(_lane_sum(q) * (1.0 / E) - mu * mu, 0.0)
                rv = _sc_rsqrt(var + 1e-5)
                for j in range(E // 16):
                    dsj = pl.ds(16 * j, 16)
                    out_buf[t, dsj] = (out_buf[t, dsj] - mu) * rv * g_v[dsj] + b_v[dsj]
                return carry2

            lax.fori_loop(0, KTOK, tok_body, 0)
            pltpu.sync_copy(out_buf, out_hbm.at[pl.ds(obase + blk * KTOK, KTOK)])

        bufs0 = (c0, l0, r0)
        bufs1 = (c1, l1, r1)
        start_gathers(0, bufs0, sem0)

        if nblk_w == 1:
            wait_gathers(bufs0, sem0)
            compute_block(0, bufs0)
            return

        def pair_body(p, carry):
            blk = 2 * p
            start_gathers(blk + 1, bufs1, sem1)
            wait_gathers(bufs0, sem0)
            compute_block(blk, bufs0)

            @pl.when(p < nblk_w // 2 - 1)
            def _():
                start_gathers(blk + 2, bufs0, sem0)

            wait_gathers(bufs1, sem1)
            compute_block(blk + 1, bufs1)
            return carry

        lax.fori_loop(0, nblk_w // 2, pair_body, 0)

    return _sc_body


def kernel(input_ids, attention_mask, table, pca_lookup, W_cont, b_cont,
           W_fuse, b_fuse, gamma, beta):
    B, S = input_ids.shape
    N = B * S
    nblk = N // TOK_BLK
    nblk_tc = nblk - SC_BLKS
    n_sc = SC_BLKS * TOK_BLK
    tok_pw = n_sc // N_WORKERS
    f32 = jnp.float32

    table_p = jnp.zeros((ROWS, E), f32).at[:table.shape[0]].set(table)
    pca_p = jnp.zeros((ROWS, 128), f32).at[:pca_lookup.shape[0], :pca_lookup.shape[1]].set(pca_lookup)
    wc_p = jnp.zeros((128, E), f32).at[:W_cont.shape[0]].set(W_cont)
    ids32 = input_ids.astype(jnp.int32).reshape(nblk, TOK_BLK)
    m32 = attention_mask.astype(jnp.int32).reshape(nblk, TOK_BLK)

    i3 = jax.ShapeDtypeStruct((nblk, 1, TOK_BLK), jnp.int32)
    (t2c, p2c, bias, idc, idcp, idl, idr, inv,
     bank, sidxc, sidxl, sidxr) = pl.pallas_call(
        _prep_body,
        out_shape=(
            jax.ShapeDtypeStruct((2 * VOCAB_PAD, E), jnp.bfloat16),
            jax.ShapeDtypeStruct((2 * VOCAB_PAD, E), jnp.bfloat16),
            jax.ShapeDtypeStruct((1, E), f32),
            i3, i3, i3, i3,
            jax.ShapeDtypeStruct((nblk, 1, TOK_BLK), f32),
            jax.ShapeDtypeStruct((NBANK, E), f32),
            i3, i3, i3,
        ),
    )(ids32, m32, table_p, pca_p, wc_p, W_fuse,
      b_cont.reshape(1, E), b_fuse.reshape(1, E))

    sc_call = functools.partial(
        pl.kernel,
        mesh=plsc.VectorSubcoreMesh(core_axis_name="c", subcore_axis_name="s"),
        out_type=jax.ShapeDtypeStruct((n_sc, E), f32),
        scratch_types=(
            [pltpu.VMEM((tok_pw,), jnp.int32)] * 3
            + [pltpu.VMEM((KTOK, E), f32)] * 7
            + [pltpu.VMEM((E,), f32)] * 2
            + [pltpu.SemaphoreType.DMA] * 2
        ),
    )(_make_sc_body(nblk_tc * TOK_BLK, tok_pw))
    sc_out = sc_call(bank, sidxc.reshape(N), sidxl.reshape(N), sidxr.reshape(N),
                     gamma, beta)

    blk_i = pl.BlockSpec((1, 1, TOK_BLK), lambda i: (i, 0, 0))
    full = lambda shape: pl.BlockSpec(shape, lambda i: (0,) * len(shape))
    tc_out = pl.pallas_call(
        _tc_body,
        grid=(nblk_tc,),
        in_specs=[blk_i, blk_i, blk_i, blk_i, blk_i,
                  full((2 * VOCAB_PAD, E)), full((2 * VOCAB_PAD, E)), full((1, E)),
                  full((1, E)), full((1, E))],
        out_specs=pl.BlockSpec((TOK_BLK, E), lambda i: (i, 0)),
        out_shape=jax.ShapeDtypeStruct((nblk_tc * TOK_BLK, E), f32),
    )(idc, idcp, idl, idr, inv, t2c, p2c, bias,
      gamma.reshape(1, E), beta.reshape(1, E))

    out = jnp.concatenate([tc_out, sc_out], axis=0)
    return out.reshape(B, S, E)
